# baseline (speedup 1.0000x reference)
import functools

import jax
import jax.numpy as jnp
from jax import lax
from jax.experimental import pallas as pl
from jax.experimental.pallas import tpu as pltpu
from jax.experimental.pallas import tpu_sc as plsc

N = 10000
NU = 2000
NI = 8000
D = 128
H3 = 64
E = 320000
NPAD = 10240
RPT = NPAD // 16
NC = 2
NS = 16
NW = NC * NS
CH = 128
NCH0 = 92
NCH1 = 65
EPT0 = NCH0 * CH
EPT1 = NCH1 * CH
EPTMAX = max(EPT0, EPT1)
EPAD = NS * (EPT0 + EPT1)
EPALLOC = EPAD + abs(EPT0 - EPT1)
L = 16

_sc_mesh = plsc.VectorSubcoreMesh(core_axis_name="c", subcore_axis_name="s")


def _sc_degree_body(dsts_hbm, out_hbm, dst_v, hist_v, red_v, res_v, hall_sh):
    c = lax.axis_index("c")
    s = lax.axis_index("s")

    def zero_hist(i, _):
        hist_v[pl.ds(i * L, L)] = jnp.zeros((L,), jnp.float32)
        return 0

    lax.fori_loop(0, NPAD // L, zero_hist, 0)
    nch = lax.select(c == 0, NCH0, NCH1)
    base0 = lax.select(c == 0, s * EPT0, NS * EPT0 + s * EPT1)
    pltpu.sync_copy(dsts_hbm.at[pl.ds(pl.multiple_of(base0, CH), EPTMAX)],
                    dst_v)
    one = jnp.ones((L,), jnp.float32)

    def body(k, _):
        idx = dst_v[pl.ds(k * L, L)]
        plsc.addupdate_scatter(hist_v, [idx], one)
        return 0

    lax.fori_loop(0, nch * (CH // L), body, 0)
    pltpu.sync_copy(hist_v, hall_sh.at[s])
    plsc.subcore_barrier()
    pltpu.sync_copy(hall_sh.at[:, pl.ds(s * RPT, RPT)], red_v)

    def red(j, _):
        acc = jnp.zeros((L,), jnp.float32)
        for r in range(NS):
            acc = acc + red_v[r, pl.ds(j * L, L)]
        res_v[pl.ds(j * L, L)] = acc
        return 0

    lax.fori_loop(0, RPT // L, red, 0)
    pltpu.sync_copy(res_v, out_hbm.at[pl.ds(c * NPAD + s * RPT, RPT)])


def _make_sc_degree(interpret=False):
    return pl.kernel(
        _sc_degree_body,
        out_type=jax.ShapeDtypeStruct((2 * NPAD,), jnp.float32),
        mesh=_sc_mesh,
        scratch_types=[
            pltpu.VMEM((EPTMAX,), jnp.int32),
            pltpu.VMEM((NPAD,), jnp.float32),
            pltpu.VMEM((NS, RPT), jnp.float32),
            pltpu.VMEM((RPT,), jnp.float32),
            pltpu.VMEM_SHARED((NS, NPAD), jnp.float32),
        ],
        compiler_params=pltpu.CompilerParams(needs_layout_passes=False),
        interpret=interpret,
    )


def _sc_scatter_body(m_hbm, srcs_hbm, dsts_hbm, out_hbm,
                     src_a, dst_a, rows_a, acc_sh, sem_ga):
    c = lax.axis_index("c")
    s = lax.axis_index("s")
    pltpu.sync_copy(m_hbm.at[pl.ds(s * RPT, RPT)],
                    acc_sh.at[pl.ds(s * RPT, RPT)])
    plsc.subcore_barrier()

    nch = lax.select(c == 0, NCH0, NCH1)
    base0 = lax.select(c == 0, s * EPT0, NS * EPT0 + s * EPT1)

    def body(k, _):
        base = pl.multiple_of(base0 + k * CH, CH)
        pltpu.sync_copy(srcs_hbm.at[pl.ds(base, CH)], src_a)
        pltpu.sync_copy(dsts_hbm.at[pl.ds(base, CH)], dst_a)
        pltpu.async_copy(m_hbm.at[src_a], rows_a, sem_ga).wait()
        pltpu.sync_copy(rows_a, acc_sh.at[dst_a], add=True)
        return 0

    lax.fori_loop(0, nch, body, 0)
    plsc.subcore_barrier()
    pltpu.sync_copy(acc_sh.at[pl.ds(s * RPT, RPT)],
                    out_hbm.at[pl.ds(c * NPAD + s * RPT, RPT)])


def _sc_scatter_body_spmem(m_hbm, srcs_hbm, dsts_hbm, out_hbm,
                           src_a, dst_a, rows_a, mst_sh, acc_sh, sem_ga):
    c = lax.axis_index("c")
    s = lax.axis_index("s")
    nch = lax.select(c == 0, NCH0, NCH1)
    base0 = lax.select(c == 0, s * EPT0, NS * EPT0 + s * EPT1)
    DH = D // 2
    for h in range(2):
        rsl = pl.ds(s * RPT, RPT)
        csl = pl.ds(h * DH, DH)
        pltpu.sync_copy(m_hbm.at[rsl, csl], mst_sh.at[rsl])
        pltpu.sync_copy(m_hbm.at[rsl, csl], acc_sh.at[rsl])
        plsc.subcore_barrier()

        def body(k, _):
            base = pl.multiple_of(base0 + k * CH, CH)
            pltpu.sync_copy(srcs_hbm.at[pl.ds(base, CH)], src_a)
            pltpu.sync_copy(dsts_hbm.at[pl.ds(base, CH)], dst_a)
            pltpu.async_copy(mst_sh.at[src_a], rows_a, sem_ga).wait()
            pltpu.sync_copy(rows_a, acc_sh.at[dst_a], add=True)
            return 0

        lax.fori_loop(0, nch, body, 0)
        plsc.subcore_barrier()
        pltpu.sync_copy(acc_sh.at[rsl],
                        out_hbm.at[pl.ds(c * NPAD + s * RPT, RPT), csl])
        plsc.subcore_barrier()


def _make_sc_scatter_spmem(interpret=False):
    return pl.kernel(
        _sc_scatter_body_spmem,
        out_type=jax.ShapeDtypeStruct((2 * NPAD, D), jnp.float32),
        mesh=_sc_mesh,
        scratch_types=[
            pltpu.VMEM((CH,), jnp.int32),
            pltpu.VMEM((CH,), jnp.int32),
            pltpu.VMEM((CH, D // 2), jnp.float32),
            pltpu.VMEM_SHARED((NPAD, D // 2), jnp.float32),
            pltpu.VMEM_SHARED((NPAD, D // 2), jnp.float32),
            pltpu.SemaphoreType.DMA,
        ],
        compiler_params=pltpu.CompilerParams(use_tc_tiling_on_sc=False),
        interpret=interpret,
    )


def _make_sc_scatter(interpret=False):
    return pl.kernel(
        _sc_scatter_body,
        out_type=jax.ShapeDtypeStruct((2 * NPAD, D), jnp.float32),
        mesh=_sc_mesh,
        scratch_types=[
            pltpu.VMEM((CH,), jnp.int32),
            pltpu.VMEM((CH,), jnp.int32),
            pltpu.VMEM((CH, D), jnp.float32),
            pltpu.VMEM_SHARED((NPAD, D), jnp.float32),
            pltpu.SemaphoreType.DMA,
        ],
        interpret=interpret,
    )


_sc_degree = _make_sc_degree()
_sc_scatter = _make_sc_scatter_spmem()


def _dinv_body(d0_ref, d1_ref, o_ref):
    o_ref[...] = lax.rsqrt(1.0 + d0_ref[...] + d1_ref[...])


def _mm_scale_body(h_ref, dinv_ref, w_ref, o_ref):
    o_ref[...] = jnp.dot(h_ref[...], w_ref[...],
                         preferred_element_type=jnp.float32) * dinv_ref[...]


def _conv_out_mm_body(s0_ref, s1_ref, m_ref, dinv_ref, b_ref, w_ref, o_ref):
    h = jnp.maximum(
        (s0_ref[...] + s1_ref[...] - m_ref[...]) * dinv_ref[...] + b_ref[...],
        0.0)
    o_ref[...] = jnp.dot(h, w_ref[...],
                         preferred_element_type=jnp.float32) * dinv_ref[...]


def _conv_out_head_body(s0_ref, s1_ref, m_ref, dinv_ref, b2_ref, w3_ref,
                        b3_ref, o_ref):
    h = jnp.maximum(
        (s0_ref[...] + s1_ref[...] - m_ref[...]) * dinv_ref[...] + b2_ref[...],
        0.0)
    o_ref[...] = jnp.maximum(
        jnp.dot(h, w3_ref[...], preferred_element_type=jnp.float32)
        + b3_ref[...], 0.0)


def _scores_body(u_ref, it_ref, o_ref):
    s = lax.dot_general(u_ref[...], it_ref[...],
                        (((1,), (1,)), ((), ())),
                        preferred_element_type=jnp.float32)
    o_ref[...] = jnp.clip(s, 1.0, 5.0)


_ROWB = 256
_NROWB = NPAD // _ROWB

_row_spec = pl.BlockSpec((_ROWB, D), lambda i: (i, 0))
_dinv_spec = pl.BlockSpec((_ROWB, 1), lambda i: (i, 0))


def _full(shape):
    return pl.BlockSpec(shape, lambda i: tuple(0 for _ in shape))


def kernel(x, edge_index, num_users, emb, W1, b1, W2, b2, W3, b3):
    f32 = jnp.float32
    src = jnp.pad(edge_index[0], (0, EPALLOC - E), constant_values=N)
    dst = jnp.pad(edge_index[1], (0, EPALLOC - E), constant_values=N)

    h0 = jnp.concatenate([
        jnp.broadcast_to(emb[0:1], (NU, D)),
        emb[1:NI + 1],
        jnp.zeros((NPAD - N, D), f32),
    ], axis=0)

    deg2 = _sc_degree(dst)
    d0 = deg2[:NPAD].reshape(NPAD // D, D)
    d1 = deg2[NPAD:].reshape(NPAD // D, D)
    dinv2d = pl.pallas_call(
        _dinv_body,
        out_shape=jax.ShapeDtypeStruct((NPAD // D, D), f32),
    )(d0, d1)
    dinv = dinv2d.reshape(NPAD, 1)

    m1 = pl.pallas_call(
        _mm_scale_body,
        grid=(_NROWB,),
        in_specs=[_row_spec, _dinv_spec, _full((D, D))],
        out_specs=_row_spec,
        out_shape=jax.ShapeDtypeStruct((NPAD, D), f32),
    )(h0, dinv, W1)
    s1 = _sc_scatter(m1, src, dst)

    m2 = pl.pallas_call(
        _conv_out_mm_body,
        grid=(_NROWB,),
        in_specs=[_row_spec, _row_spec, _row_spec, _dinv_spec,
                  _full((1, D)), _full((D, D))],
        out_specs=_row_spec,
        out_shape=jax.ShapeDtypeStruct((NPAD, D), f32),
    )(s1[:NPAD], s1[NPAD:], m1, dinv, b1.reshape(1, D), W2)
    s2 = _sc_scatter(m2, src, dst)

    h3 = pl.pallas_call(
        _conv_out_head_body,
        grid=(_NROWB,),
        in_specs=[_row_spec, _row_spec, _row_spec, _dinv_spec,
                  _full((1, D)), _full((D, H3)), _full((1, H3))],
        out_specs=pl.BlockSpec((_ROWB, H3), lambda i: (i, 0)),
        out_shape=jax.ShapeDtypeStruct((NPAD, H3), f32),
    )(s2[:NPAD], s2[NPAD:], m2, dinv, b2.reshape(1, D), W3, b3.reshape(1, H3))

    u = h3[:NU]
    it = h3[NU:N]
    MB, NB = 512, 512
    scores = pl.pallas_call(
        _scores_body,
        grid=(pl.cdiv(NU, MB), pl.cdiv(NI, NB)),
        in_specs=[pl.BlockSpec((MB, H3), lambda i, j: (i, 0)),
                  pl.BlockSpec((NB, H3), lambda i, j: (j, 0))],
        out_specs=pl.BlockSpec((MB, NB), lambda i, j: (i, j)),
        out_shape=jax.ShapeDtypeStruct((NU, NI), f32),
    )(u, it)
    return scores


# speedup vs baseline: 1.1826x; 1.1826x over previous
import functools

import jax
import jax.numpy as jnp
from jax import lax
from jax.experimental import pallas as pl
from jax.experimental.pallas import tpu as pltpu
from jax.experimental.pallas import tpu_sc as plsc

N = 10000
NU = 2000
NI = 8000
D = 128
H3 = 64
E = 320000
NPAD = 10240
RPT = NPAD // 16
NC = 2
NS = 16
NW = NC * NS
CH = 128
NCH0 = 92
NCH1 = 65
EPT0 = NCH0 * CH
EPT1 = NCH1 * CH
EPTMAX = max(EPT0, EPT1)
EPAD = NS * (EPT0 + EPT1)
EPALLOC = EPAD + abs(EPT0 - EPT1)
L = 16

_sc_mesh = plsc.VectorSubcoreMesh(core_axis_name="c", subcore_axis_name="s")


def _sc_degree_body(dsts_hbm, out_hbm, dst_v, hist_v, red_v, res_v, hall_sh):
    c = lax.axis_index("c")
    s = lax.axis_index("s")

    def zero_hist(i, _):
        hist_v[pl.ds(i * L, L)] = jnp.zeros((L,), jnp.float32)
        return 0

    lax.fori_loop(0, NPAD // L, zero_hist, 0)
    nch = lax.select(c == 0, NCH0, NCH1)
    base0 = lax.select(c == 0, s * EPT0, NS * EPT0 + s * EPT1)
    pltpu.sync_copy(dsts_hbm.at[pl.ds(pl.multiple_of(base0, CH), EPTMAX)],
                    dst_v)
    one = jnp.ones((L,), jnp.float32)

    def body(k, _):
        idx = dst_v[pl.ds(k * L, L)]
        plsc.addupdate_scatter(hist_v, [idx], one)
        return 0

    lax.fori_loop(0, nch * (CH // L), body, 0)
    pltpu.sync_copy(hist_v, hall_sh.at[s])
    plsc.subcore_barrier()
    pltpu.sync_copy(hall_sh.at[:, pl.ds(s * RPT, RPT)], red_v)

    def red(j, _):
        acc = jnp.zeros((L,), jnp.float32)
        for r in range(NS):
            acc = acc + red_v[r, pl.ds(j * L, L)]
        res_v[pl.ds(j * L, L)] = acc
        return 0

    lax.fori_loop(0, RPT // L, red, 0)
    pltpu.sync_copy(res_v, out_hbm.at[pl.ds(c * NPAD + s * RPT, RPT)])


def _make_sc_degree(interpret=False):
    return pl.kernel(
        _sc_degree_body,
        out_type=jax.ShapeDtypeStruct((2 * NPAD,), jnp.float32),
        mesh=_sc_mesh,
        scratch_types=[
            pltpu.VMEM((EPTMAX,), jnp.int32),
            pltpu.VMEM((NPAD,), jnp.float32),
            pltpu.VMEM((NS, RPT), jnp.float32),
            pltpu.VMEM((RPT,), jnp.float32),
            pltpu.VMEM_SHARED((NS, NPAD), jnp.float32),
        ],
        compiler_params=pltpu.CompilerParams(needs_layout_passes=False),
        interpret=interpret,
    )


def _sc_scatter_body(m_hbm, srcs_hbm, dsts_hbm, out_hbm,
                     src_a, dst_a, rows_a, acc_sh, sem_ga):
    c = lax.axis_index("c")
    s = lax.axis_index("s")
    pltpu.sync_copy(m_hbm.at[pl.ds(s * RPT, RPT)],
                    acc_sh.at[pl.ds(s * RPT, RPT)])
    plsc.subcore_barrier()

    nch = lax.select(c == 0, NCH0, NCH1)
    base0 = lax.select(c == 0, s * EPT0, NS * EPT0 + s * EPT1)

    def body(k, _):
        base = pl.multiple_of(base0 + k * CH, CH)
        pltpu.sync_copy(srcs_hbm.at[pl.ds(base, CH)], src_a)
        pltpu.sync_copy(dsts_hbm.at[pl.ds(base, CH)], dst_a)
        pltpu.async_copy(m_hbm.at[src_a], rows_a, sem_ga).wait()
        pltpu.sync_copy(rows_a, acc_sh.at[dst_a], add=True)
        return 0

    lax.fori_loop(0, nch, body, 0)
    plsc.subcore_barrier()
    pltpu.sync_copy(acc_sh.at[pl.ds(s * RPT, RPT)],
                    out_hbm.at[pl.ds(c * NPAD + s * RPT, RPT)])


def _sc_scatter_body_spmem(m_hbm, srcs_hbm, dsts_hbm, out_hbm,
                           src_a, dst_a, rows_a, mst_sh, acc_sh, sem_ga):
    c = lax.axis_index("c")
    s = lax.axis_index("s")
    nch = lax.select(c == 0, NCH0, NCH1)
    base0 = lax.select(c == 0, s * EPT0, NS * EPT0 + s * EPT1)
    DH = D // 2
    for h in range(2):
        rsl = pl.ds(s * RPT, RPT)
        csl = pl.ds(h * DH, DH)
        pltpu.sync_copy(m_hbm.at[rsl, csl], mst_sh.at[rsl])
        pltpu.sync_copy(m_hbm.at[rsl, csl], acc_sh.at[rsl])
        plsc.subcore_barrier()

        def body(k, _):
            base = pl.multiple_of(base0 + k * CH, CH)
            pltpu.sync_copy(srcs_hbm.at[pl.ds(base, CH)], src_a)
            pltpu.sync_copy(dsts_hbm.at[pl.ds(base, CH)], dst_a)
            pltpu.async_copy(mst_sh.at[src_a], rows_a, sem_ga).wait()
            pltpu.sync_copy(rows_a, acc_sh.at[dst_a], add=True)
            return 0

        lax.fori_loop(0, nch, body, 0)
        plsc.subcore_barrier()
        pltpu.sync_copy(acc_sh.at[rsl],
                        out_hbm.at[pl.ds(c * NPAD + s * RPT, RPT), csl])
        plsc.subcore_barrier()


def _make_sc_scatter_spmem(interpret=False):
    return pl.kernel(
        _sc_scatter_body_spmem,
        out_type=jax.ShapeDtypeStruct((2 * NPAD, D), jnp.float32),
        mesh=_sc_mesh,
        scratch_types=[
            pltpu.VMEM((CH,), jnp.int32),
            pltpu.VMEM((CH,), jnp.int32),
            pltpu.VMEM((CH, D // 2), jnp.float32),
            pltpu.VMEM_SHARED((NPAD, D // 2), jnp.float32),
            pltpu.VMEM_SHARED((NPAD, D // 2), jnp.float32),
            pltpu.SemaphoreType.DMA,
        ],
        compiler_params=pltpu.CompilerParams(use_tc_tiling_on_sc=False),
        interpret=interpret,
    )


def _make_sc_scatter(interpret=False):
    return pl.kernel(
        _sc_scatter_body,
        out_type=jax.ShapeDtypeStruct((2 * NPAD, D), jnp.float32),
        mesh=_sc_mesh,
        scratch_types=[
            pltpu.VMEM((CH,), jnp.int32),
            pltpu.VMEM((CH,), jnp.int32),
            pltpu.VMEM((CH, D), jnp.float32),
            pltpu.VMEM_SHARED((NPAD, D), jnp.float32),
            pltpu.SemaphoreType.DMA,
        ],
        interpret=interpret,
    )


_sc_degree = _make_sc_degree()
_sc_scatter = _make_sc_scatter()


def _mm_scale_body(h_ref, d0_ref, d1_ref, w_ref, o_ref):
    dinv = lax.rsqrt(1.0 + d0_ref[...] + d1_ref[...])
    o_ref[...] = jnp.dot(h_ref[...], w_ref[...],
                         preferred_element_type=jnp.float32) * dinv


def _conv_out_mm_body(s0_ref, s1_ref, m_ref, d0_ref, d1_ref, b_ref, w_ref,
                      o_ref):
    dinv = lax.rsqrt(1.0 + d0_ref[...] + d1_ref[...])
    h = jnp.maximum(
        (s0_ref[...] + s1_ref[...] - m_ref[...]) * dinv + b_ref[...], 0.0)
    o_ref[...] = jnp.dot(h, w_ref[...],
                         preferred_element_type=jnp.float32) * dinv


def _conv_out_head_body(s0_ref, s1_ref, m_ref, d0_ref, d1_ref, b2_ref, w3_ref,
                        b3_ref, o_ref):
    dinv = lax.rsqrt(1.0 + d0_ref[...] + d1_ref[...])
    h = jnp.maximum(
        (s0_ref[...] + s1_ref[...] - m_ref[...]) * dinv + b2_ref[...], 0.0)
    o_ref[...] = jnp.maximum(
        jnp.dot(h, w3_ref[...], preferred_element_type=jnp.float32)
        + b3_ref[...], 0.0)


def _scores_body(u_ref, it_ref, o_ref):
    s = lax.dot_general(u_ref[...], it_ref[...],
                        (((1,), (1,)), ((), ())),
                        preferred_element_type=jnp.float32)
    o_ref[...] = jnp.clip(s, 1.0, 5.0)


_ROWB = 256
_NROWB = NPAD // _ROWB

_row_spec = pl.BlockSpec((_ROWB, D), lambda i: (i, 0))
_dinv_spec = pl.BlockSpec((_ROWB, 1), lambda i: (i, 0))


def _full(shape):
    return pl.BlockSpec(shape, lambda i: tuple(0 for _ in shape))


def kernel(x, edge_index, num_users, emb, W1, b1, W2, b2, W3, b3):
    f32 = jnp.float32
    src = jnp.pad(edge_index[0], (0, EPALLOC - E), constant_values=N)
    dst = jnp.pad(edge_index[1], (0, EPALLOC - E), constant_values=N)

    h0 = jnp.concatenate([
        jnp.broadcast_to(emb[0:1], (NU, D)),
        emb[1:NI + 1],
        jnp.zeros((NPAD - N, D), f32),
    ], axis=0)

    deg2 = _sc_degree(dst)
    d0 = deg2[:NPAD].reshape(NPAD, 1)
    d1 = deg2[NPAD:].reshape(NPAD, 1)

    m1 = pl.pallas_call(
        _mm_scale_body,
        grid=(_NROWB,),
        in_specs=[_row_spec, _dinv_spec, _dinv_spec, _full((D, D))],
        out_specs=_row_spec,
        out_shape=jax.ShapeDtypeStruct((NPAD, D), f32),
    )(h0, d0, d1, W1)
    s1 = _sc_scatter(m1, src, dst)

    m2 = pl.pallas_call(
        _conv_out_mm_body,
        grid=(_NROWB,),
        in_specs=[_row_spec, _row_spec, _row_spec, _dinv_spec, _dinv_spec,
                  _full((1, D)), _full((D, D))],
        out_specs=_row_spec,
        out_shape=jax.ShapeDtypeStruct((NPAD, D), f32),
    )(s1[:NPAD], s1[NPAD:], m1, d0, d1, b1.reshape(1, D), W2)
    s2 = _sc_scatter(m2, src, dst)

    h3 = pl.pallas_call(
        _conv_out_head_body,
        grid=(_NROWB,),
        in_specs=[_row_spec, _row_spec, _row_spec, _dinv_spec, _dinv_spec,
                  _full((1, D)), _full((D, H3)), _full((1, H3))],
        out_specs=pl.BlockSpec((_ROWB, H3), lambda i: (i, 0)),
        out_shape=jax.ShapeDtypeStruct((NPAD, H3), f32),
    )(s2[:NPAD], s2[NPAD:], m2, d0, d1, b2.reshape(1, D), W3,
      b3.reshape(1, H3))

    u = h3[:NU]
    it = h3[NU:N]
    MB, NB = 512, 512
    scores = pl.pallas_call(
        _scores_body,
        grid=(pl.cdiv(NU, MB), pl.cdiv(NI, NB)),
        in_specs=[pl.BlockSpec((MB, H3), lambda i, j: (i, 0)),
                  pl.BlockSpec((NB, H3), lambda i, j: (j, 0))],
        out_specs=pl.BlockSpec((MB, NB), lambda i, j: (i, j)),
        out_shape=jax.ShapeDtypeStruct((NU, NI), f32),
    )(u, it)
    return scores


# speedup vs baseline: 1.2004x; 1.0150x over previous
import functools

import jax
import jax.numpy as jnp
from jax import lax
from jax.experimental import pallas as pl
from jax.experimental.pallas import tpu as pltpu
from jax.experimental.pallas import tpu_sc as plsc

N = 10000
NU = 2000
NI = 8000
D = 128
H3 = 64
E = 320000
NPAD = 10240
RPT = NPAD // 16
NC = 2
NS = 16
NW = NC * NS
CH = 128
NCH0 = 92
NCH1 = 65
EPT0 = NCH0 * CH
EPT1 = NCH1 * CH
EPTMAX = max(EPT0, EPT1)
EPAD = NS * (EPT0 + EPT1)
EPALLOC = EPAD + abs(EPT0 - EPT1)
L = 16

_sc_mesh = plsc.VectorSubcoreMesh(core_axis_name="c", subcore_axis_name="s")


def _sc_degree_body(dsts_hbm, out_hbm, dst_v, hist_v, red_v, res_v, hall_sh):
    c = lax.axis_index("c")
    s = lax.axis_index("s")

    def zero_hist(i, _):
        hist_v[pl.ds(i * L, L)] = jnp.zeros((L,), jnp.float32)
        return 0

    lax.fori_loop(0, NPAD // L, zero_hist, 0)
    nch = lax.select(c == 0, NCH0, NCH1)
    base0 = lax.select(c == 0, s * EPT0, NS * EPT0 + s * EPT1)
    pltpu.sync_copy(dsts_hbm.at[pl.ds(pl.multiple_of(base0, CH), EPTMAX)],
                    dst_v)
    one = jnp.ones((L,), jnp.float32)

    def body(k, _):
        idx = dst_v[pl.ds(k * L, L)]
        plsc.addupdate_scatter(hist_v, [idx], one)
        return 0

    lax.fori_loop(0, nch * (CH // L), body, 0)
    pltpu.sync_copy(hist_v, hall_sh.at[s])
    plsc.subcore_barrier()
    pltpu.sync_copy(hall_sh.at[:, pl.ds(s * RPT, RPT)], red_v)

    def red(j, _):
        acc = jnp.zeros((L,), jnp.float32)
        for r in range(NS):
            acc = acc + red_v[r, pl.ds(j * L, L)]
        res_v[pl.ds(j * L, L)] = acc
        return 0

    lax.fori_loop(0, RPT // L, red, 0)
    pltpu.sync_copy(res_v, out_hbm.at[pl.ds(c * NPAD + s * RPT, RPT)])


def _make_sc_degree(interpret=False):
    return pl.kernel(
        _sc_degree_body,
        out_type=jax.ShapeDtypeStruct((2 * NPAD,), jnp.float32),
        mesh=_sc_mesh,
        scratch_types=[
            pltpu.VMEM((EPTMAX,), jnp.int32),
            pltpu.VMEM((NPAD,), jnp.float32),
            pltpu.VMEM((NS, RPT), jnp.float32),
            pltpu.VMEM((RPT,), jnp.float32),
            pltpu.VMEM_SHARED((NS, NPAD), jnp.float32),
        ],
        compiler_params=pltpu.CompilerParams(needs_layout_passes=False),
        interpret=interpret,
    )


def _sc_scatter_body(m_hbm, srcs_hbm, dsts_hbm, out_hbm,
                     src_a, dst_a, rows_a, acc_sh, sem_ga):
    c = lax.axis_index("c")
    s = lax.axis_index("s")
    pltpu.sync_copy(m_hbm.at[pl.ds(s * RPT, RPT)],
                    acc_sh.at[pl.ds(s * RPT, RPT)])
    plsc.subcore_barrier()

    nch = lax.select(c == 0, NCH0, NCH1)
    base0 = lax.select(c == 0, s * EPT0, NS * EPT0 + s * EPT1)

    def body(k, _):
        base = pl.multiple_of(base0 + k * CH, CH)
        pltpu.sync_copy(srcs_hbm.at[pl.ds(base, CH)], src_a)
        pltpu.sync_copy(dsts_hbm.at[pl.ds(base, CH)], dst_a)
        pltpu.async_copy(m_hbm.at[src_a], rows_a, sem_ga).wait()
        pltpu.sync_copy(rows_a, acc_sh.at[dst_a], add=True)
        return 0

    lax.fori_loop(0, nch, body, 0)
    plsc.subcore_barrier()
    pltpu.sync_copy(acc_sh.at[pl.ds(s * RPT, RPT)],
                    out_hbm.at[pl.ds(c * NPAD + s * RPT, RPT)])


def _sc_scatter_body_spmem(m_hbm, srcs_hbm, dsts_hbm, out_hbm,
                           src_a, dst_a, rows_a, mst_sh, acc_sh, sem_ga):
    c = lax.axis_index("c")
    s = lax.axis_index("s")
    nch = lax.select(c == 0, NCH0, NCH1)
    base0 = lax.select(c == 0, s * EPT0, NS * EPT0 + s * EPT1)
    DH = D // 2
    for h in range(2):
        rsl = pl.ds(s * RPT, RPT)
        csl = pl.ds(h * DH, DH)
        pltpu.sync_copy(m_hbm.at[rsl, csl], mst_sh.at[rsl])
        pltpu.sync_copy(m_hbm.at[rsl, csl], acc_sh.at[rsl])
        plsc.subcore_barrier()

        def body(k, _):
            base = pl.multiple_of(base0 + k * CH, CH)
            pltpu.sync_copy(srcs_hbm.at[pl.ds(base, CH)], src_a)
            pltpu.sync_copy(dsts_hbm.at[pl.ds(base, CH)], dst_a)
            pltpu.async_copy(mst_sh.at[src_a], rows_a, sem_ga).wait()
            pltpu.sync_copy(rows_a, acc_sh.at[dst_a], add=True)
            return 0

        lax.fori_loop(0, nch, body, 0)
        plsc.subcore_barrier()
        pltpu.sync_copy(acc_sh.at[rsl],
                        out_hbm.at[pl.ds(c * NPAD + s * RPT, RPT), csl])
        plsc.subcore_barrier()


def _make_sc_scatter_spmem(interpret=False):
    return pl.kernel(
        _sc_scatter_body_spmem,
        out_type=jax.ShapeDtypeStruct((2 * NPAD, D), jnp.float32),
        mesh=_sc_mesh,
        scratch_types=[
            pltpu.VMEM((CH,), jnp.int32),
            pltpu.VMEM((CH,), jnp.int32),
            pltpu.VMEM((CH, D // 2), jnp.float32),
            pltpu.VMEM_SHARED((NPAD, D // 2), jnp.float32),
            pltpu.VMEM_SHARED((NPAD, D // 2), jnp.float32),
            pltpu.SemaphoreType.DMA,
        ],
        compiler_params=pltpu.CompilerParams(use_tc_tiling_on_sc=False),
        interpret=interpret,
    )


def _make_sc_scatter(interpret=False):
    return pl.kernel(
        _sc_scatter_body,
        out_type=jax.ShapeDtypeStruct((2 * NPAD, D), jnp.float32),
        mesh=_sc_mesh,
        scratch_types=[
            pltpu.VMEM((CH,), jnp.int32),
            pltpu.VMEM((CH,), jnp.int32),
            pltpu.VMEM((CH, D), jnp.float32),
            pltpu.VMEM_SHARED((NPAD, D), jnp.float32),
            pltpu.SemaphoreType.DMA,
        ],
        interpret=interpret,
    )


_sc_degree = _make_sc_degree()
_sc_scatter = _make_sc_scatter()


def _mm_scale_body(h_ref, d0_ref, d1_ref, w_ref, o_ref):
    dinv = lax.rsqrt(1.0 + d0_ref[...] + d1_ref[...])
    o_ref[...] = jnp.dot(h_ref[...], w_ref[...],
                         preferred_element_type=jnp.float32) * dinv


def _conv_out_mm_body(s0_ref, s1_ref, m_ref, d0_ref, d1_ref, b_ref, w_ref,
                      o_ref):
    dinv = lax.rsqrt(1.0 + d0_ref[...] + d1_ref[...])
    h = jnp.maximum(
        (s0_ref[0] + s1_ref[0] - m_ref[...]) * dinv + b_ref[...], 0.0)
    o_ref[...] = jnp.dot(h, w_ref[...],
                         preferred_element_type=jnp.float32) * dinv


def _conv_out_head_body(s0_ref, s1_ref, m_ref, d0_ref, d1_ref, b2_ref, w3_ref,
                        b3_ref, o_ref):
    dinv = lax.rsqrt(1.0 + d0_ref[...] + d1_ref[...])
    h = jnp.maximum(
        (s0_ref[0] + s1_ref[0] - m_ref[...]) * dinv + b2_ref[...], 0.0)
    o_ref[...] = jnp.maximum(
        jnp.dot(h, w3_ref[...], preferred_element_type=jnp.float32)
        + b3_ref[...], 0.0)


def _scores_body(u_ref, it_ref, o_ref):
    s = lax.dot_general(u_ref[...], it_ref[...],
                        (((1,), (1,)), ((), ())),
                        preferred_element_type=jnp.float32)
    o_ref[...] = jnp.clip(s, 1.0, 5.0)


_ROWB = 256
_NROWB = NPAD // _ROWB

_row_spec = pl.BlockSpec((_ROWB, D), lambda i: (i, 0))
_dinv_spec = pl.BlockSpec((_ROWB, 1), lambda i: (i, 0))


def _full(shape):
    return pl.BlockSpec(shape, lambda i: tuple(0 for _ in shape))


def kernel(x, edge_index, num_users, emb, W1, b1, W2, b2, W3, b3):
    f32 = jnp.float32
    src = jnp.pad(edge_index[0], (0, EPALLOC - E), constant_values=N)
    dst = jnp.pad(edge_index[1], (0, EPALLOC - E), constant_values=N)

    h0 = jnp.concatenate([
        jnp.broadcast_to(emb[0:1], (NU, D)),
        emb[1:NI + 1],
        jnp.zeros((NPAD - N, D), f32),
    ], axis=0)

    deg2 = _sc_degree(dst)
    d0 = deg2[:NPAD].reshape(NPAD, 1)
    d1 = deg2[NPAD:].reshape(NPAD, 1)

    m1 = pl.pallas_call(
        _mm_scale_body,
        grid=(_NROWB,),
        in_specs=[_row_spec, _dinv_spec, _dinv_spec, _full((D, D))],
        out_specs=_row_spec,
        out_shape=jax.ShapeDtypeStruct((NPAD, D), f32),
    )(h0, d0, d1, W1)
    s1 = _sc_scatter(m1, src, dst).reshape(2, NPAD, D)

    _p0_spec = pl.BlockSpec((1, _ROWB, D), lambda i: (0, i, 0))
    _p1_spec = pl.BlockSpec((1, _ROWB, D), lambda i: (1, i, 0))

    m2 = pl.pallas_call(
        _conv_out_mm_body,
        grid=(_NROWB,),
        in_specs=[_p0_spec, _p1_spec, _row_spec, _dinv_spec, _dinv_spec,
                  _full((1, D)), _full((D, D))],
        out_specs=_row_spec,
        out_shape=jax.ShapeDtypeStruct((NPAD, D), f32),
    )(s1, s1, m1, d0, d1, b1.reshape(1, D), W2)
    s2 = _sc_scatter(m2, src, dst).reshape(2, NPAD, D)

    h3 = pl.pallas_call(
        _conv_out_head_body,
        grid=(_NROWB,),
        in_specs=[_p0_spec, _p1_spec, _row_spec, _dinv_spec, _dinv_spec,
                  _full((1, D)), _full((D, H3)), _full((1, H3))],
        out_specs=pl.BlockSpec((_ROWB, H3), lambda i: (i, 0)),
        out_shape=jax.ShapeDtypeStruct((NPAD, H3), f32),
    )(s2, s2, m2, d0, d1, b2.reshape(1, D), W3, b3.reshape(1, H3))

    u = h3[:NU]
    it = h3[NU:N]
    MB, NB = 512, 512
    scores = pl.pallas_call(
        _scores_body,
        grid=(pl.cdiv(NU, MB), pl.cdiv(NI, NB)),
        in_specs=[pl.BlockSpec((MB, H3), lambda i, j: (i, 0)),
                  pl.BlockSpec((NB, H3), lambda i, j: (j, 0))],
        out_specs=pl.BlockSpec((MB, NB), lambda i, j: (i, j)),
        out_shape=jax.ShapeDtypeStruct((NU, NI), f32),
    )(u, it)
    return scores


# speedup vs baseline: 1.5032x; 1.2522x over previous
import functools

import jax
import jax.numpy as jnp
from jax import lax
from jax.experimental import pallas as pl
from jax.experimental.pallas import tpu as pltpu
from jax.experimental.pallas import tpu_sc as plsc

N = 10000
NU = 2000
NI = 8000
D = 128
H3 = 64
E = 320000
NPAD = 10240
RPT = NPAD // 16
NC = 2
NS = 16
NW = NC * NS
CH = 128
NCH0 = 92
NCH1 = 65
EPT0 = NCH0 * CH
EPT1 = NCH1 * CH
EPTMAX = max(EPT0, EPT1)
EPAD = NS * (EPT0 + EPT1)
EPALLOC = EPAD + abs(EPT0 - EPT1)
L = 16

_sc_mesh = plsc.VectorSubcoreMesh(core_axis_name="c", subcore_axis_name="s")


def _sc_degree_body(dsts_hbm, out_hbm, dst_v, hist_v, red_v, res_v, hall_sh):
    c = lax.axis_index("c")
    s = lax.axis_index("s")

    def zero_hist(i, _):
        hist_v[pl.ds(i * L, L)] = jnp.zeros((L,), jnp.float32)
        return 0

    lax.fori_loop(0, NPAD // L, zero_hist, 0)
    nch = lax.select(c == 0, NCH0, NCH1)
    base0 = lax.select(c == 0, s * EPT0, NS * EPT0 + s * EPT1)
    pltpu.sync_copy(dsts_hbm.at[pl.ds(pl.multiple_of(base0, CH), EPTMAX)],
                    dst_v)
    one = jnp.ones((L,), jnp.float32)

    def body(k, _):
        idx = dst_v[pl.ds(k * L, L)]
        plsc.addupdate_scatter(hist_v, [idx], one)
        return 0

    lax.fori_loop(0, nch * (CH // L), body, 0)
    pltpu.sync_copy(hist_v, hall_sh.at[s])
    plsc.subcore_barrier()
    pltpu.sync_copy(hall_sh.at[:, pl.ds(s * RPT, RPT)], red_v)

    def red(j, _):
        acc = jnp.zeros((L,), jnp.float32)
        for r in range(NS):
            acc = acc + red_v[r, pl.ds(j * L, L)]
        res_v[pl.ds(j * L, L)] = acc
        return 0

    lax.fori_loop(0, RPT // L, red, 0)
    pltpu.sync_copy(res_v, out_hbm.at[pl.ds(c * NPAD + s * RPT, RPT)])


def _make_sc_degree(interpret=False):
    return pl.kernel(
        _sc_degree_body,
        out_type=jax.ShapeDtypeStruct((2 * NPAD,), jnp.float32),
        mesh=_sc_mesh,
        scratch_types=[
            pltpu.VMEM((EPTMAX,), jnp.int32),
            pltpu.VMEM((NPAD,), jnp.float32),
            pltpu.VMEM((NS, RPT), jnp.float32),
            pltpu.VMEM((RPT,), jnp.float32),
            pltpu.VMEM_SHARED((NS, NPAD), jnp.float32),
        ],
        compiler_params=pltpu.CompilerParams(needs_layout_passes=False),
        interpret=interpret,
    )


NCHMAX = max(NCH0, NCH1)


def _sc_scatter_body(m_hbm, srcs_hbm, dsts_hbm, out_hbm,
                     src_all, dst_all, rows_a, acc_sh, sem_ga):
    c = lax.axis_index("c")
    s = lax.axis_index("s")
    pltpu.sync_copy(m_hbm.at[pl.ds(s * RPT, RPT)],
                    acc_sh.at[pl.ds(s * RPT, RPT)])

    nch = lax.select(c == 0, NCH0, NCH1)
    chunk0 = lax.select(c == 0, s * NCH0, NS * NCH0 + s * NCH1)
    pltpu.sync_copy(srcs_hbm.at[pl.ds(chunk0, NCHMAX)], src_all)
    pltpu.sync_copy(dsts_hbm.at[pl.ds(chunk0, NCHMAX)], dst_all)
    plsc.subcore_barrier()

    def body(k, _):
        pltpu.async_copy(m_hbm.at[src_all.at[k]], rows_a, sem_ga).wait()
        pltpu.sync_copy(rows_a, acc_sh.at[dst_all.at[k]], add=True)
        return 0

    lax.fori_loop(0, nch, body, 0)
    plsc.subcore_barrier()
    pltpu.sync_copy(acc_sh.at[pl.ds(s * RPT, RPT)],
                    out_hbm.at[pl.ds(c * NPAD + s * RPT, RPT)])


def _sc_scatter_body_spmem(m_hbm, srcs_hbm, dsts_hbm, out_hbm,
                           src_a, dst_a, rows_a, mst_sh, acc_sh, sem_ga):
    c = lax.axis_index("c")
    s = lax.axis_index("s")
    nch = lax.select(c == 0, NCH0, NCH1)
    base0 = lax.select(c == 0, s * EPT0, NS * EPT0 + s * EPT1)
    DH = D // 2
    for h in range(2):
        rsl = pl.ds(s * RPT, RPT)
        csl = pl.ds(h * DH, DH)
        pltpu.sync_copy(m_hbm.at[rsl, csl], mst_sh.at[rsl])
        pltpu.sync_copy(m_hbm.at[rsl, csl], acc_sh.at[rsl])
        plsc.subcore_barrier()

        def body(k, _):
            base = pl.multiple_of(base0 + k * CH, CH)
            pltpu.sync_copy(srcs_hbm.at[pl.ds(base, CH)], src_a)
            pltpu.sync_copy(dsts_hbm.at[pl.ds(base, CH)], dst_a)
            pltpu.async_copy(mst_sh.at[src_a], rows_a, sem_ga).wait()
            pltpu.sync_copy(rows_a, acc_sh.at[dst_a], add=True)
            return 0

        lax.fori_loop(0, nch, body, 0)
        plsc.subcore_barrier()
        pltpu.sync_copy(acc_sh.at[rsl],
                        out_hbm.at[pl.ds(c * NPAD + s * RPT, RPT), csl])
        plsc.subcore_barrier()


def _make_sc_scatter_spmem(interpret=False):
    return pl.kernel(
        _sc_scatter_body_spmem,
        out_type=jax.ShapeDtypeStruct((2 * NPAD, D), jnp.float32),
        mesh=_sc_mesh,
        scratch_types=[
            pltpu.VMEM((CH,), jnp.int32),
            pltpu.VMEM((CH,), jnp.int32),
            pltpu.VMEM((CH, D // 2), jnp.float32),
            pltpu.VMEM_SHARED((NPAD, D // 2), jnp.float32),
            pltpu.VMEM_SHARED((NPAD, D // 2), jnp.float32),
            pltpu.SemaphoreType.DMA,
        ],
        compiler_params=pltpu.CompilerParams(use_tc_tiling_on_sc=False),
        interpret=interpret,
    )


def _make_sc_scatter(interpret=False):
    return pl.kernel(
        _sc_scatter_body,
        out_type=jax.ShapeDtypeStruct((2 * NPAD, D), jnp.float32),
        mesh=_sc_mesh,
        scratch_types=[
            pltpu.VMEM((NCHMAX, CH), jnp.int32),
            pltpu.VMEM((NCHMAX, CH), jnp.int32),
            pltpu.VMEM((CH, D), jnp.float32),
            pltpu.VMEM_SHARED((NPAD, D), jnp.float32),
            pltpu.SemaphoreType.DMA,
        ],
        compiler_params=pltpu.CompilerParams(use_tc_tiling_on_sc=False),
        interpret=interpret,
    )


_sc_degree = _make_sc_degree()
_sc_scatter = _make_sc_scatter()


def _mm_scale_body(h_ref, d0_ref, d1_ref, w_ref, o_ref):
    dinv = lax.rsqrt(1.0 + d0_ref[...] + d1_ref[...])
    o_ref[...] = jnp.dot(h_ref[...], w_ref[...],
                         preferred_element_type=jnp.float32) * dinv


def _conv_out_mm_body(s0_ref, s1_ref, m_ref, d0_ref, d1_ref, b_ref, w_ref,
                      o_ref):
    dinv = lax.rsqrt(1.0 + d0_ref[...] + d1_ref[...])
    h = jnp.maximum(
        (s0_ref[0] + s1_ref[0] - m_ref[...]) * dinv + b_ref[...], 0.0)
    o_ref[...] = jnp.dot(h, w_ref[...],
                         preferred_element_type=jnp.float32) * dinv


def _conv_out_head_body(s0_ref, s1_ref, m_ref, d0_ref, d1_ref, b2_ref, w3_ref,
                        b3_ref, o_ref):
    dinv = lax.rsqrt(1.0 + d0_ref[...] + d1_ref[...])
    h = jnp.maximum(
        (s0_ref[0] + s1_ref[0] - m_ref[...]) * dinv + b2_ref[...], 0.0)
    o_ref[...] = jnp.maximum(
        jnp.dot(h, w3_ref[...], preferred_element_type=jnp.float32)
        + b3_ref[...], 0.0)


def _scores_body(u_ref, it_ref, o_ref):
    s = lax.dot_general(u_ref[...], it_ref[...],
                        (((1,), (1,)), ((), ())),
                        preferred_element_type=jnp.float32)
    o_ref[...] = jnp.clip(s, 1.0, 5.0)


_ROWB = 256
_NROWB = NPAD // _ROWB

_row_spec = pl.BlockSpec((_ROWB, D), lambda i: (i, 0))
_dinv_spec = pl.BlockSpec((_ROWB, 1), lambda i: (i, 0))


def _full(shape):
    return pl.BlockSpec(shape, lambda i: tuple(0 for _ in shape))


def kernel(x, edge_index, num_users, emb, W1, b1, W2, b2, W3, b3):
    f32 = jnp.float32
    src = jnp.pad(edge_index[0], (0, EPALLOC - E), constant_values=N)
    dst = jnp.pad(edge_index[1], (0, EPALLOC - E), constant_values=N)
    src2d = src.reshape(EPALLOC // CH, CH)
    dst2d = dst.reshape(EPALLOC // CH, CH)

    h0 = jnp.concatenate([
        jnp.broadcast_to(emb[0:1], (NU, D)),
        emb[1:NI + 1],
        jnp.zeros((NPAD - N, D), f32),
    ], axis=0)

    deg2 = _sc_degree(dst)
    d0 = deg2[:NPAD].reshape(NPAD, 1)
    d1 = deg2[NPAD:].reshape(NPAD, 1)

    m1 = pl.pallas_call(
        _mm_scale_body,
        grid=(_NROWB,),
        in_specs=[_row_spec, _dinv_spec, _dinv_spec, _full((D, D))],
        out_specs=_row_spec,
        out_shape=jax.ShapeDtypeStruct((NPAD, D), f32),
    )(h0, d0, d1, W1)
    s1 = _sc_scatter(m1, src2d, dst2d).reshape(2, NPAD, D)

    _p0_spec = pl.BlockSpec((1, _ROWB, D), lambda i: (0, i, 0))
    _p1_spec = pl.BlockSpec((1, _ROWB, D), lambda i: (1, i, 0))

    m2 = pl.pallas_call(
        _conv_out_mm_body,
        grid=(_NROWB,),
        in_specs=[_p0_spec, _p1_spec, _row_spec, _dinv_spec, _dinv_spec,
                  _full((1, D)), _full((D, D))],
        out_specs=_row_spec,
        out_shape=jax.ShapeDtypeStruct((NPAD, D), f32),
    )(s1, s1, m1, d0, d1, b1.reshape(1, D), W2)
    s2 = _sc_scatter(m2, src2d, dst2d).reshape(2, NPAD, D)

    h3 = pl.pallas_call(
        _conv_out_head_body,
        grid=(_NROWB,),
        in_specs=[_p0_spec, _p1_spec, _row_spec, _dinv_spec, _dinv_spec,
                  _full((1, D)), _full((D, H3)), _full((1, H3))],
        out_specs=pl.BlockSpec((_ROWB, H3), lambda i: (i, 0)),
        out_shape=jax.ShapeDtypeStruct((NPAD, H3), f32),
    )(s2, s2, m2, d0, d1, b2.reshape(1, D), W3, b3.reshape(1, H3))

    u = h3[:NU]
    it = h3[NU:N]
    MB, NB = 512, 512
    scores = pl.pallas_call(
        _scores_body,
        grid=(pl.cdiv(NU, MB), pl.cdiv(NI, NB)),
        in_specs=[pl.BlockSpec((MB, H3), lambda i, j: (i, 0)),
                  pl.BlockSpec((NB, H3), lambda i, j: (j, 0))],
        out_specs=pl.BlockSpec((MB, NB), lambda i, j: (i, j)),
        out_shape=jax.ShapeDtypeStruct((NU, NI), f32),
    )(u, it)
    return scores
